# baseline (speedup 1.0000x reference)
import jax
import jax.numpy as jnp
from jax.experimental import pallas as pl
from jax.experimental.pallas import tpu as pltpu

_N = 4096
_BLK = 512
_NB = _N // _BLK


def _gcn_fused_kernel(x_ref, a_ref, w1_ref, b1_ref, w2_ref, b2_ref,
                      w3_ref, b3_ref, wf_ref, bf_ref, out_ref,
                      a_bf, h1, h2):
    i = pl.program_id(1)
    f32 = jnp.float32
    bf16 = jnp.bfloat16

    def _linear(v, w_ref, b_ref):
        return jax.lax.dot_general(
            v.astype(bf16), w_ref[:].astype(bf16),
            (((1,), (1,)), ((), ())),
            preferred_element_type=f32) + b_ref[:]

    a_blk = a_ref[0].astype(bf16)
    a_bf[pl.ds(i * _BLK, _BLK), :] = a_blk
    xb = x_ref[0]
    ns = jnp.dot(a_blk, xb.astype(bf16), preferred_element_type=f32)
    comb = x_ref[0, pl.ds(i * _BLK, _BLK), :] + ns
    h1[pl.ds(i * _BLK, _BLK), :] = jax.nn.relu(_linear(comb, w1_ref, b1_ref))

    @pl.when(i == _NB - 1)
    def _tail():
        for hin, hout, w_ref, b_ref in ((h1, h2, w2_ref, b2_ref),
                                        (h2, h1, w3_ref, b3_ref)):
            hb = hin[:].astype(bf16)

            def body(j, _):
                aj = a_bf[pl.ds(j * _BLK, _BLK), :]
                nsj = jnp.dot(aj, hb, preferred_element_type=f32)
                combj = hin[pl.ds(j * _BLK, _BLK), :] + nsj
                hout[pl.ds(j * _BLK, _BLK), :] = jax.nn.relu(
                    _linear(combj, w_ref, b_ref))
                return 0

            jax.lax.fori_loop(0, _NB, body, 0)

        pooled = jnp.max(h1[:], axis=0, keepdims=True)
        out_ref[0] = _linear(pooled, wf_ref, bf_ref)


def kernel(x, edge_index, adjacency, W1, b1, W2, b2, W3, b3, Wf, bf):
    del edge_index
    B, N, Fin = x.shape
    H = W1.shape[0]
    C = Wf.shape[0]

    grid = (B, _NB)
    out = pl.pallas_call(
        _gcn_fused_kernel,
        grid=grid,
        in_specs=[
            pl.BlockSpec((1, N, Fin), lambda b, i: (b, 0, 0)),
            pl.BlockSpec((1, _BLK, N), lambda b, i: (b, i, 0)),
            pl.BlockSpec((H, Fin), lambda b, i: (0, 0)),
            pl.BlockSpec((1, H), lambda b, i: (0, 0)),
            pl.BlockSpec((H, H), lambda b, i: (0, 0)),
            pl.BlockSpec((1, H), lambda b, i: (0, 0)),
            pl.BlockSpec((H, H), lambda b, i: (0, 0)),
            pl.BlockSpec((1, H), lambda b, i: (0, 0)),
            pl.BlockSpec((C, H), lambda b, i: (0, 0)),
            pl.BlockSpec((1, C), lambda b, i: (0, 0)),
        ],
        out_specs=pl.BlockSpec((1, 1, C), lambda b, i: (b, 0, 0)),
        out_shape=jax.ShapeDtypeStruct((B, 1, C), jnp.float32),
        scratch_shapes=[
            pltpu.VMEM((N, N), jnp.bfloat16),
            pltpu.VMEM((N, H), jnp.float32),
            pltpu.VMEM((N, H), jnp.float32),
        ],
        compiler_params=pltpu.CompilerParams(
            dimension_semantics=("parallel", "arbitrary"),
            vmem_limit_bytes=112 * 1024 * 1024,
        ),
    )(x, adjacency, W1, b1.reshape(1, H), W2, b2.reshape(1, H),
      W3, b3.reshape(1, H), Wf, bf.reshape(1, C))
    return out.reshape(B, C)


# speedup vs baseline: 1.1307x; 1.1307x over previous
import jax
import jax.numpy as jnp
from jax.experimental import pallas as pl
from jax.experimental.pallas import tpu as pltpu

_N = 4096
_BLK = 512
_NB = _N // _BLK


def _gcn_fused_kernel(x_ref, a_ref, w1_ref, b1_ref, w2_ref, b2_ref,
                      w3_ref, b3_ref, wf_ref, bf_ref, out_ref,
                      a_bf, h1, h2):
    i = pl.program_id(1)
    f32 = jnp.float32
    bf16 = jnp.bfloat16

    def _linear(v, w_ref, b_ref):
        return jax.lax.dot_general(
            v.astype(bf16), w_ref[:].astype(bf16),
            (((1,), (1,)), ((), ())),
            preferred_element_type=f32) + b_ref[:]

    a_blk = a_ref[0].astype(bf16)
    a_bf[pl.ds(i * _BLK, _BLK), :] = a_blk
    xb = x_ref[0]
    ns = jnp.dot(a_blk, xb.astype(bf16), preferred_element_type=f32)
    comb = x_ref[0, pl.ds(i * _BLK, _BLK), :] + ns
    h1[pl.ds(i * _BLK, _BLK), :] = jax.nn.relu(_linear(comb, w1_ref, b1_ref))

    @pl.when(i == _NB - 1)
    def _tail():
        for hin, hout, w_ref, b_ref in ((h1, h2, w2_ref, b2_ref),
                                        (h2, h1, w3_ref, b3_ref)):
            hb = hin[:].astype(bf16)
            ns = jnp.dot(a_bf[:], hb, preferred_element_type=f32)
            hout[:] = jax.nn.relu(_linear(hin[:] + ns, w_ref, b_ref))

        pooled = jnp.max(h1[:], axis=0, keepdims=True)
        out_ref[0] = _linear(pooled, wf_ref, bf_ref)


def kernel(x, edge_index, adjacency, W1, b1, W2, b2, W3, b3, Wf, bf):
    del edge_index
    B, N, Fin = x.shape
    H = W1.shape[0]
    C = Wf.shape[0]

    grid = (B, _NB)
    out = pl.pallas_call(
        _gcn_fused_kernel,
        grid=grid,
        in_specs=[
            pl.BlockSpec((1, N, Fin), lambda b, i: (b, 0, 0)),
            pl.BlockSpec((1, _BLK, N), lambda b, i: (b, i, 0)),
            pl.BlockSpec((H, Fin), lambda b, i: (0, 0)),
            pl.BlockSpec((1, H), lambda b, i: (0, 0)),
            pl.BlockSpec((H, H), lambda b, i: (0, 0)),
            pl.BlockSpec((1, H), lambda b, i: (0, 0)),
            pl.BlockSpec((H, H), lambda b, i: (0, 0)),
            pl.BlockSpec((1, H), lambda b, i: (0, 0)),
            pl.BlockSpec((C, H), lambda b, i: (0, 0)),
            pl.BlockSpec((1, C), lambda b, i: (0, 0)),
        ],
        out_specs=pl.BlockSpec((1, 1, C), lambda b, i: (b, 0, 0)),
        out_shape=jax.ShapeDtypeStruct((B, 1, C), jnp.float32),
        scratch_shapes=[
            pltpu.VMEM((N, N), jnp.bfloat16),
            pltpu.VMEM((N, H), jnp.float32),
            pltpu.VMEM((N, H), jnp.float32),
        ],
        compiler_params=pltpu.CompilerParams(
            dimension_semantics=("parallel", "arbitrary"),
            vmem_limit_bytes=112 * 1024 * 1024,
        ),
    )(x, adjacency, W1, b1.reshape(1, H), W2, b2.reshape(1, H),
      W3, b3.reshape(1, H), Wf, bf.reshape(1, C))
    return out.reshape(B, C)
